# baseline (speedup 1.0000x reference)
import jax
import jax.numpy as jnp
from jax.experimental import pallas as pl
from jax.experimental.pallas import tpu as pltpu

_B, _C, _T, _F = 16, 2, 2000, 257
_FB = 136
_NF = 2
_NSTEP = 11


def _ema_norm_kernel(xr_ref, xi_ref, s1_ref, a_ref, p_ref, dec_ref,
                     w_ref, b_ref, res_ref, sm_ref):
    xr0 = xr_ref[0, 0, 0]
    xi0 = xi_ref[0, 0, 0]
    xr1 = xr_ref[0, 0, 1]
    xi1 = xi_ref[0, 0, 1]

    d2_0 = xr0 * xr0 + xi0 * xi0
    d2_1 = xr1 * xr1 + xi1 * xi1

    y = d2_0
    d = 1
    for k in range(_NSTEP):
        y = y + dec_ref[k] * jnp.roll(y, d, axis=1)
        d *= 2

    a_bc = jnp.broadcast_to(a_ref[:, 0:1], (_FB, _T))
    s1_bc = jnp.broadcast_to(s1_ref[0, :, 0:1], (_FB, _T))
    s = a_bc * y + p_ref[...] * s1_bc

    smooth = jnp.sqrt(s)
    sm_ref[0] = smooth

    w0 = jnp.broadcast_to(w_ref[0, :, 0:1], (_FB, _T))
    w1 = jnp.broadcast_to(w_ref[1, :, 0:1], (_FB, _T))
    b0 = jnp.broadcast_to(b_ref[0, :, 0:1], (_FB, _T))
    b1 = jnp.broadcast_to(b_ref[1, :, 0:1], (_FB, _T))
    inv0 = 1.0 / (smooth + 1e-8) * w0
    inv1 = 1.0 / (jnp.sqrt(d2_1) + 1e-8) * w1
    res_ref[0, 0, 0] = xr0 * inv0 + b0
    res_ref[1, 0, 0] = xi0 * inv0 + b0
    res_ref[0, 0, 1] = xr1 * inv1 + b1
    res_ref[1, 0, 1] = xi1 * inv1 + b1


def kernel(input, s_1, weights, bias, alpha_param):
    B, C, T, F = _B, _C, _T, _F

    xp = input.transpose(4, 0, 1, 3, 2)

    a = jax.nn.sigmoid(alpha_param.reshape(F))
    la = jnp.log1p(-a)
    liota = jnp.arange(T, dtype=jnp.float32)
    p_d = jnp.exp(la[:, None] * (liota[None, :] + 1.0))
    decs = []
    d = 1
    for _ in range(_NSTEP):
        decs.append(jnp.where(liota[None, :] >= d,
                              jnp.exp(la * float(d))[:, None], 0.0))
        d *= 2
    dec_d = jnp.stack(decs, axis=0)

    a_b = jnp.broadcast_to(a[:, None], (F, 128))
    s1_b = jnp.broadcast_to(s_1.reshape(B, F, 1), (B, F, 128))
    w_b = jnp.broadcast_to(weights.reshape(C, F, 1), (C, F, 128))
    b_b = jnp.broadcast_to(bias.reshape(C, F, 1), (C, F, 128))

    resp, smooth = pl.pallas_call(
        _ema_norm_kernel,
        grid=(_NF, B),
        in_specs=[
            pl.BlockSpec((1, 1, C, _FB, T), lambda f, b: (0, b, 0, f, 0)),
            pl.BlockSpec((1, 1, C, _FB, T), lambda f, b: (1, b, 0, f, 0)),
            pl.BlockSpec((1, _FB, 128), lambda f, b: (b, f, 0)),
            pl.BlockSpec((_FB, 128), lambda f, b: (f, 0)),
            pl.BlockSpec((_FB, T), lambda f, b: (f, 0)),
            pl.BlockSpec((_NSTEP, _FB, T), lambda f, b: (0, f, 0)),
            pl.BlockSpec((C, _FB, 128), lambda f, b: (0, f, 0)),
            pl.BlockSpec((C, _FB, 128), lambda f, b: (0, f, 0)),
        ],
        out_specs=[
            pl.BlockSpec((2, 1, C, _FB, T), lambda f, b: (0, b, 0, f, 0)),
            pl.BlockSpec((1, _FB, T), lambda f, b: (b, f, 0)),
        ],
        out_shape=[
            jax.ShapeDtypeStruct((2, B, C, F, T), jnp.float32),
            jax.ShapeDtypeStruct((B, F, T), jnp.float32),
        ],
        compiler_params=pltpu.CompilerParams(
            dimension_semantics=("parallel", "parallel"),
            vmem_limit_bytes=56 * 1024 * 1024,
        ),
        name="ema_norm",
    )(xp, xp, s1_b, a_b, p_d, dec_d, w_b, b_b)

    res = resp.transpose(1, 2, 4, 3, 0)
    smooth_data = smooth.transpose(0, 2, 1).reshape(B, 1, T, F, 1)
    s_final = (smooth[:, :, T - 1] ** 2).reshape(B, 1, F, 1)
    return res, s_final, smooth_data


# speedup vs baseline: 1.0601x; 1.0601x over previous
import jax
import jax.numpy as jnp
from jax.experimental import pallas as pl
from jax.experimental.pallas import tpu as pltpu

_B, _C, _T, _F = 16, 2, 2000, 257
_TB = 1024
_NT = -(-_T // _TB)
_NSTEP = 10


def _ema_norm_kernel(xr_ref, xi_ref, s1_ref, a_ref, p_ref, dec_ref,
                     w_ref, b_ref, res_ref, sm_ref, carry_ref):
    t = pl.program_id(1)

    @pl.when(t == 0)
    def _():
        carry_ref[...] = pltpu.repeat(s1_ref[0], _TB // 128, axis=1)

    xr0 = xr_ref[0, 0, 0]
    xi0 = xi_ref[0, 0, 0]
    xr1 = xr_ref[0, 0, 1]
    xi1 = xi_ref[0, 0, 1]

    d2_0 = xr0 * xr0 + xi0 * xi0
    d2_1 = xr1 * xr1 + xi1 * xi1

    liota = jax.lax.broadcasted_iota(jnp.int32, (_F, _TB), 1)
    y = jnp.where(liota < _T - t * _TB, d2_0 * a_ref[...], 0.0)
    d = 1
    for k in range(_NSTEP):
        y = y + dec_ref[k] * jnp.roll(y, d, axis=1)
        d *= 2

    s = y + p_ref[...] * carry_ref[...]
    carry_ref[...] = jnp.broadcast_to(s[:, _TB - 1:_TB], s.shape)

    smooth = jnp.sqrt(s)
    sm_ref[0] = smooth

    wr = pltpu.repeat(w_ref[...], _TB // 128, axis=2)
    br = pltpu.repeat(b_ref[...], _TB // 128, axis=2)
    inv0 = 1.0 / (smooth + 1e-8) * wr[0]
    inv1 = 1.0 / (jnp.sqrt(d2_1) + 1e-8) * wr[1]
    res_ref[0, 0, 0] = xr0 * inv0 + br[0]
    res_ref[1, 0, 0] = xi0 * inv0 + br[0]
    res_ref[0, 0, 1] = xr1 * inv1 + br[1]
    res_ref[1, 0, 1] = xi1 * inv1 + br[1]


def kernel(input, s_1, weights, bias, alpha_param):
    B, C, T, F, TB = _B, _C, _T, _F, _TB

    xp = input.transpose(4, 0, 1, 3, 2)

    a = jax.nn.sigmoid(alpha_param.reshape(F))
    la = jnp.log1p(-a)
    liota = jnp.arange(TB, dtype=jnp.float32)
    p_d = jnp.exp(la[:, None] * (liota[None, :] + 1.0))
    decs = []
    d = 1
    for _ in range(_NSTEP):
        decs.append(jnp.where(liota[None, :] >= d,
                              jnp.exp(la * float(d))[:, None], 0.0))
        d *= 2
    dec_d = jnp.stack(decs, axis=0)

    a_full = jnp.broadcast_to(a[:, None], (F, TB))
    s1_b = jnp.broadcast_to(s_1.reshape(B, F, 1), (B, F, 128))
    w_b = jnp.broadcast_to(weights.reshape(C, F, 1), (C, F, 128))
    b_b = jnp.broadcast_to(bias.reshape(C, F, 1), (C, F, 128))

    resp, smooth = pl.pallas_call(
        _ema_norm_kernel,
        grid=(B, _NT),
        in_specs=[
            pl.BlockSpec((1, 1, C, F, TB), lambda b, t: (0, b, 0, 0, t)),
            pl.BlockSpec((1, 1, C, F, TB), lambda b, t: (1, b, 0, 0, t)),
            pl.BlockSpec((1, F, 128), lambda b, t: (b, 0, 0)),
            pl.BlockSpec((F, TB), lambda b, t: (0, 0)),
            pl.BlockSpec((F, TB), lambda b, t: (0, 0)),
            pl.BlockSpec((_NSTEP, F, TB), lambda b, t: (0, 0, 0)),
            pl.BlockSpec((C, F, 128), lambda b, t: (0, 0, 0)),
            pl.BlockSpec((C, F, 128), lambda b, t: (0, 0, 0)),
        ],
        out_specs=[
            pl.BlockSpec((2, 1, C, F, TB), lambda b, t: (0, b, 0, 0, t)),
            pl.BlockSpec((1, F, TB), lambda b, t: (b, 0, t)),
        ],
        out_shape=[
            jax.ShapeDtypeStruct((2, B, C, F, T), jnp.float32),
            jax.ShapeDtypeStruct((B, F, T), jnp.float32),
        ],
        scratch_shapes=[pltpu.VMEM((_F, TB), jnp.float32)],
        compiler_params=pltpu.CompilerParams(
            dimension_semantics=("parallel", "arbitrary"),
            vmem_limit_bytes=56 * 1024 * 1024,
        ),
        name="ema_norm",
    )(xp, xp, s1_b, a_full, p_d, dec_d, w_b, b_b)

    res = resp.transpose(1, 2, 4, 3, 0)
    smooth_data = smooth.transpose(0, 2, 1).reshape(B, 1, T, F, 1)
    s_final = (smooth[:, :, T - 1] ** 2).reshape(B, 1, F, 1)
    return res, s_final, smooth_data
